# initial kernel scaffold (unmeasured)
import jax
import jax.numpy as jnp
from jax import lax
from jax.experimental import pallas as pl
from jax.experimental.pallas import tpu as pltpu

N_DEV = 4


def _gelu(y):
    c = 0.7978845608028654
    return 0.5 * y * (1.0 + jnp.tanh(c * (y + 0.044715 * y * y * y)))


def kernel(x, w_mat):
    m, k_per = x.shape
    _, n = w_mat.shape
    ch = m // N_DEV

    def body(x_ref, w_ref, out_ref, rs_buf, send_sems, recv_sems):
        my = lax.axis_index("i")
        left = (my - 1) % N_DEV
        right = (my + 1) % N_DEV

        barrier_sem = pltpu.get_barrier_semaphore()
        for nbr in (left, right):
            pl.semaphore_signal(
                barrier_sem, inc=1,
                device_id=(nbr,), device_id_type=pl.DeviceIdType.MESH,
            )
        pl.semaphore_wait(barrier_sem, 2)

        out_ref[:, :] = jnp.dot(
            x_ref[:, :], w_ref[:, :], preferred_element_type=jnp.float32
        )

        for h in range(N_DEV - 1):
            send_chunk = (my - h) % N_DEV
            recv_chunk = (my - h - 1) % N_DEV
            rdma = pltpu.make_async_remote_copy(
                src_ref=out_ref.at[pl.ds(send_chunk * ch, ch), :],
                dst_ref=rs_buf.at[h],
                send_sem=send_sems.at[h],
                recv_sem=recv_sems.at[h],
                device_id=(right,),
                device_id_type=pl.DeviceIdType.MESH,
            )
            rdma.start()
            rdma.wait()
            acc = out_ref[pl.ds(recv_chunk * ch, ch), :] + rs_buf[h, :, :]
            out_ref[pl.ds(recv_chunk * ch, ch), :] = acc

        own = (my + 1) % N_DEV
        out_ref[pl.ds(own * ch, ch), :] = _gelu(out_ref[pl.ds(own * ch, ch), :])

        for g in range(N_DEV - 1):
            send_chunk = (my + 1 - g) % N_DEV
            rdma = pltpu.make_async_remote_copy(
                src_ref=out_ref.at[pl.ds(send_chunk * ch, ch), :],
                dst_ref=out_ref.at[pl.ds(send_chunk * ch, ch), :],
                send_sem=send_sems.at[N_DEV - 1 + g],
                recv_sem=recv_sems.at[N_DEV - 1 + g],
                device_id=(right,),
                device_id_type=pl.DeviceIdType.MESH,
            )
            rdma.start()
            rdma.wait()

    return pl.pallas_call(
        body,
        out_shape=jax.ShapeDtypeStruct((m, n), jnp.float32),
        in_specs=[
            pl.BlockSpec(memory_space=pltpu.VMEM),
            pl.BlockSpec(memory_space=pltpu.VMEM),
        ],
        out_specs=pl.BlockSpec(memory_space=pltpu.VMEM),
        scratch_shapes=[
            pltpu.VMEM((N_DEV - 1, ch, n), jnp.float32),
            pltpu.SemaphoreType.DMA((2 * (N_DEV - 1),)),
            pltpu.SemaphoreType.DMA((2 * (N_DEV - 1),)),
        ],
        compiler_params=pltpu.CompilerParams(collective_id=0),
    )(x, w_mat)


# baseline (device time: 313207 ns/iter reference)
import jax
import jax.numpy as jnp
from jax import lax
from jax.experimental import pallas as pl
from jax.experimental.pallas import tpu as pltpu

N_DEV = 4


def _gelu(y):
    c = 0.7978845608028654
    return 0.5 * y * (1.0 + jnp.tanh(c * (y + 0.044715 * y * y * y)))


def kernel(x, w_mat):
    m, k_per = x.shape
    _, n = w_mat.shape
    ch = m // N_DEV

    def body(x_ref, w_ref, out_ref, rs_buf, send_sems, recv_sems):
        my = lax.axis_index("i")
        left = (my - 1) % N_DEV
        right = (my + 1) % N_DEV

        barrier_sem = pltpu.get_barrier_semaphore()
        for nbr in (left, right):
            pl.semaphore_signal(
                barrier_sem, inc=1,
                device_id=(nbr,), device_id_type=pl.DeviceIdType.MESH,
            )
        pl.semaphore_wait(barrier_sem, 2)

        out_ref[:, :] = jnp.dot(
            x_ref[:, :], w_ref[:, :], preferred_element_type=jnp.float32
        )

        for h in range(N_DEV - 1):
            send_chunk = (my - h) % N_DEV
            recv_chunk = (my - h - 1) % N_DEV
            rdma = pltpu.make_async_remote_copy(
                src_ref=out_ref.at[pl.ds(send_chunk * ch, ch), :],
                dst_ref=rs_buf.at[h],
                send_sem=send_sems.at[h],
                recv_sem=recv_sems.at[h],
                device_id=(right,),
                device_id_type=pl.DeviceIdType.MESH,
            )
            rdma.start()
            rdma.wait()
            acc = out_ref[pl.ds(recv_chunk * ch, ch), :] + rs_buf[h, :, :]
            out_ref[pl.ds(recv_chunk * ch, ch), :] = acc

        own = (my + 1) % N_DEV
        out_ref[pl.ds(own * ch, ch), :] = _gelu(out_ref[pl.ds(own * ch, ch), :])

        for g in range(N_DEV - 1):
            send_chunk = (my + 1 - g) % N_DEV
            rdma = pltpu.make_async_remote_copy(
                src_ref=out_ref.at[pl.ds(send_chunk * ch, ch), :],
                dst_ref=out_ref.at[pl.ds(send_chunk * ch, ch), :],
                send_sem=send_sems.at[N_DEV - 1 + g],
                recv_sem=recv_sems.at[N_DEV - 1 + g],
                device_id=(right,),
                device_id_type=pl.DeviceIdType.MESH,
            )
            rdma.start()
            rdma.wait()

    return pl.pallas_call(
        body,
        out_shape=jax.ShapeDtypeStruct((m, n), jnp.float32),
        in_specs=[
            pl.BlockSpec(memory_space=pltpu.VMEM),
            pl.BlockSpec(memory_space=pltpu.VMEM),
        ],
        out_specs=pl.BlockSpec(memory_space=pltpu.VMEM),
        scratch_shapes=[
            pltpu.VMEM((N_DEV - 1, ch, n), jnp.float32),
            pltpu.SemaphoreType.DMA((2 * (N_DEV - 1),)),
            pltpu.SemaphoreType.DMA((2 * (N_DEV - 1),)),
        ],
        compiler_params=pltpu.CompilerParams(
            collective_id=0, vmem_limit_bytes=56 * 1024 * 1024
        ),
    )(x, w_mat)


# device time: 178793 ns/iter; 1.7518x vs baseline; 1.7518x over previous
import jax
import jax.numpy as jnp
from jax import lax
from jax.experimental import pallas as pl
from jax.experimental.pallas import tpu as pltpu

N_DEV = 4


def _gelu(y):
    c = 0.7978845608028654
    return 0.5 * y * (1.0 + jnp.tanh(c * (y + 0.044715 * y * y * y)))


def kernel(x, w_mat):
    m, k_per = x.shape
    _, n = w_mat.shape
    ch = m // N_DEV
    hn = n // 2

    def body(
        x_ref, w_ref, out_ref,
        rs_cw, rs_ccw, ss_cw, rs_sem_cw, ss_ccw, rs_sem_ccw,
    ):
        my = lax.axis_index("i")
        left = (my - 1) % N_DEV
        right = (my + 1) % N_DEV

        barrier_sem = pltpu.get_barrier_semaphore()
        for nbr in (left, right):
            pl.semaphore_signal(
                barrier_sem, inc=1,
                device_id=(nbr,), device_id_type=pl.DeviceIdType.MESH,
            )
        pl.semaphore_wait(barrier_sem, 2)

        out_ref[:, :] = jnp.dot(
            x_ref[:, :], w_ref[:, :], preferred_element_type=jnp.float32
        )

        for h in range(N_DEV - 1):
            cw_send = (my - h) % N_DEV
            cw_recv = (my - h - 1) % N_DEV
            ccw_send = (my + h) % N_DEV
            ccw_recv = (my + h + 1) % N_DEV
            r_cw = pltpu.make_async_remote_copy(
                src_ref=out_ref.at[pl.ds(cw_send * ch, ch), pl.ds(0, hn)],
                dst_ref=rs_cw.at[h],
                send_sem=ss_cw.at[h],
                recv_sem=rs_sem_cw.at[h],
                device_id=(right,),
                device_id_type=pl.DeviceIdType.MESH,
            )
            r_ccw = pltpu.make_async_remote_copy(
                src_ref=out_ref.at[pl.ds(ccw_send * ch, ch), pl.ds(hn, hn)],
                dst_ref=rs_ccw.at[h],
                send_sem=ss_ccw.at[h],
                recv_sem=rs_sem_ccw.at[h],
                device_id=(left,),
                device_id_type=pl.DeviceIdType.MESH,
            )
            r_cw.start()
            r_ccw.start()
            r_cw.wait()
            r_ccw.wait()
            out_ref[pl.ds(cw_recv * ch, ch), pl.ds(0, hn)] = (
                out_ref[pl.ds(cw_recv * ch, ch), pl.ds(0, hn)] + rs_cw[h, :, :]
            )
            out_ref[pl.ds(ccw_recv * ch, ch), pl.ds(hn, hn)] = (
                out_ref[pl.ds(ccw_recv * ch, ch), pl.ds(hn, hn)] + rs_ccw[h, :, :]
            )

        own_cw = (my + 1) % N_DEV
        own_ccw = (my - 1) % N_DEV
        out_ref[pl.ds(own_cw * ch, ch), pl.ds(0, hn)] = _gelu(
            out_ref[pl.ds(own_cw * ch, ch), pl.ds(0, hn)]
        )
        out_ref[pl.ds(own_ccw * ch, ch), pl.ds(hn, hn)] = _gelu(
            out_ref[pl.ds(own_ccw * ch, ch), pl.ds(hn, hn)]
        )

        for g in range(N_DEV - 1):
            cw_send = (my + 1 - g) % N_DEV
            ccw_send = (my - 1 + g) % N_DEV
            r_cw = pltpu.make_async_remote_copy(
                src_ref=out_ref.at[pl.ds(cw_send * ch, ch), pl.ds(0, hn)],
                dst_ref=out_ref.at[pl.ds(cw_send * ch, ch), pl.ds(0, hn)],
                send_sem=ss_cw.at[N_DEV - 1 + g],
                recv_sem=rs_sem_cw.at[N_DEV - 1 + g],
                device_id=(right,),
                device_id_type=pl.DeviceIdType.MESH,
            )
            r_ccw = pltpu.make_async_remote_copy(
                src_ref=out_ref.at[pl.ds(ccw_send * ch, ch), pl.ds(hn, hn)],
                dst_ref=out_ref.at[pl.ds(ccw_send * ch, ch), pl.ds(hn, hn)],
                send_sem=ss_ccw.at[N_DEV - 1 + g],
                recv_sem=rs_sem_ccw.at[N_DEV - 1 + g],
                device_id=(left,),
                device_id_type=pl.DeviceIdType.MESH,
            )
            r_cw.start()
            r_ccw.start()
            r_cw.wait()
            r_ccw.wait()

    n_sems = 2 * (N_DEV - 1)
    return pl.pallas_call(
        body,
        out_shape=jax.ShapeDtypeStruct((m, n), jnp.float32),
        in_specs=[
            pl.BlockSpec(memory_space=pltpu.VMEM),
            pl.BlockSpec(memory_space=pltpu.VMEM),
        ],
        out_specs=pl.BlockSpec(memory_space=pltpu.VMEM),
        scratch_shapes=[
            pltpu.VMEM((N_DEV - 1, ch, hn), jnp.float32),
            pltpu.VMEM((N_DEV - 1, ch, hn), jnp.float32),
            pltpu.SemaphoreType.DMA((n_sems,)),
            pltpu.SemaphoreType.DMA((n_sems,)),
            pltpu.SemaphoreType.DMA((n_sems,)),
            pltpu.SemaphoreType.DMA((n_sems,)),
        ],
        compiler_params=pltpu.CompilerParams(
            collective_id=0, vmem_limit_bytes=56 * 1024 * 1024
        ),
    )(x, w_mat)


# device time: 175507 ns/iter; 1.7846x vs baseline; 1.0187x over previous
import jax
import jax.numpy as jnp
from jax import lax
from jax.experimental import pallas as pl
from jax.experimental.pallas import tpu as pltpu

N_DEV = 4


def _gelu(y):
    c = 0.7978845608028654
    return 0.5 * y * (1.0 + jnp.tanh(c * (y + 0.044715 * y * y * y)))


def kernel(x, w_mat):
    m, k_per = x.shape
    _, n = w_mat.shape
    ch = m // N_DEV
    hn = n // 2

    def body(
        x_ref, w_ref, out_ref,
        rs_cw, rs_ccw, ss_cw, rs_sem_cw, ss_ccw, rs_sem_ccw,
    ):
        my = lax.axis_index("i")
        left = (my - 1) % N_DEV
        right = (my + 1) % N_DEV

        barrier_sem = pltpu.get_barrier_semaphore()
        for nbr in (left, right):
            pl.semaphore_signal(
                barrier_sem, inc=1,
                device_id=(nbr,), device_id_type=pl.DeviceIdType.MESH,
            )
        pl.semaphore_wait(barrier_sem, 2)

        def mm_chunk(idx):
            out_ref[pl.ds(idx * ch, ch), :] = jnp.dot(
                x_ref[pl.ds(idx * ch, ch), :], w_ref[:, :],
                preferred_element_type=jnp.float32,
            )

        def make_rs(h, cw_send, ccw_send):
            r_cw = pltpu.make_async_remote_copy(
                src_ref=out_ref.at[pl.ds(cw_send * ch, ch), pl.ds(0, hn)],
                dst_ref=rs_cw.at[h],
                send_sem=ss_cw.at[h],
                recv_sem=rs_sem_cw.at[h],
                device_id=(right,),
                device_id_type=pl.DeviceIdType.MESH,
            )
            r_ccw = pltpu.make_async_remote_copy(
                src_ref=out_ref.at[pl.ds(ccw_send * ch, ch), pl.ds(hn, hn)],
                dst_ref=rs_ccw.at[h],
                send_sem=ss_ccw.at[h],
                recv_sem=rs_sem_ccw.at[h],
                device_id=(left,),
                device_id_type=pl.DeviceIdType.MESH,
            )
            return r_cw, r_ccw

        mm_chunk(my)
        r_cw, r_ccw = make_rs(0, my, my)
        r_cw.start()
        r_ccw.start()
        mm_chunk((my - 1) % N_DEV)
        mm_chunk((my + 1) % N_DEV)
        mm_chunk((my + 2) % N_DEV)

        for h in range(N_DEV - 1):
            cw_recv = (my - h - 1) % N_DEV
            ccw_recv = (my + h + 1) % N_DEV
            r_cw.wait()
            out_ref[pl.ds(cw_recv * ch, ch), pl.ds(0, hn)] = (
                out_ref[pl.ds(cw_recv * ch, ch), pl.ds(0, hn)] + rs_cw[h, :, :]
            )
            r_ccw.wait()
            out_ref[pl.ds(ccw_recv * ch, ch), pl.ds(hn, hn)] = (
                out_ref[pl.ds(ccw_recv * ch, ch), pl.ds(hn, hn)] + rs_ccw[h, :, :]
            )
            if h < N_DEV - 2:
                r_cw, r_ccw = make_rs(h + 1, cw_recv, ccw_recv)
                r_cw.start()
                r_ccw.start()

        own_cw = (my + 1) % N_DEV
        own_ccw = (my - 1) % N_DEV
        out_ref[pl.ds(own_cw * ch, ch), pl.ds(0, hn)] = _gelu(
            out_ref[pl.ds(own_cw * ch, ch), pl.ds(0, hn)]
        )
        out_ref[pl.ds(own_ccw * ch, ch), pl.ds(hn, hn)] = _gelu(
            out_ref[pl.ds(own_ccw * ch, ch), pl.ds(hn, hn)]
        )

        for g in range(N_DEV - 1):
            cw_send = (my + 1 - g) % N_DEV
            ccw_send = (my - 1 + g) % N_DEV
            r_cw = pltpu.make_async_remote_copy(
                src_ref=out_ref.at[pl.ds(cw_send * ch, ch), pl.ds(0, hn)],
                dst_ref=out_ref.at[pl.ds(cw_send * ch, ch), pl.ds(0, hn)],
                send_sem=ss_cw.at[N_DEV - 1 + g],
                recv_sem=rs_sem_cw.at[N_DEV - 1 + g],
                device_id=(right,),
                device_id_type=pl.DeviceIdType.MESH,
            )
            r_ccw = pltpu.make_async_remote_copy(
                src_ref=out_ref.at[pl.ds(ccw_send * ch, ch), pl.ds(hn, hn)],
                dst_ref=out_ref.at[pl.ds(ccw_send * ch, ch), pl.ds(hn, hn)],
                send_sem=ss_ccw.at[N_DEV - 1 + g],
                recv_sem=rs_sem_ccw.at[N_DEV - 1 + g],
                device_id=(left,),
                device_id_type=pl.DeviceIdType.MESH,
            )
            r_cw.start()
            r_ccw.start()
            r_cw.wait()
            r_ccw.wait()

    n_sems = 2 * (N_DEV - 1)
    return pl.pallas_call(
        body,
        out_shape=jax.ShapeDtypeStruct((m, n), jnp.float32),
        in_specs=[
            pl.BlockSpec(memory_space=pltpu.VMEM),
            pl.BlockSpec(memory_space=pltpu.VMEM),
        ],
        out_specs=pl.BlockSpec(memory_space=pltpu.VMEM),
        scratch_shapes=[
            pltpu.VMEM((N_DEV - 1, ch, hn), jnp.float32),
            pltpu.VMEM((N_DEV - 1, ch, hn), jnp.float32),
            pltpu.SemaphoreType.DMA((n_sems,)),
            pltpu.SemaphoreType.DMA((n_sems,)),
            pltpu.SemaphoreType.DMA((n_sems,)),
            pltpu.SemaphoreType.DMA((n_sems,)),
        ],
        compiler_params=pltpu.CompilerParams(
            collective_id=0, vmem_limit_bytes=56 * 1024 * 1024
        ),
    )(x, w_mat)


# device time: 175486 ns/iter; 1.7848x vs baseline; 1.0001x over previous
import jax
import jax.numpy as jnp
from jax import lax
from jax.experimental import pallas as pl
from jax.experimental.pallas import tpu as pltpu

N_DEV = 4


def _gelu(y):
    c = 0.7978845608028654
    return 0.5 * y * (1.0 + jnp.tanh(c * (y + 0.044715 * y * y * y)))


def kernel(x, w_mat):
    m, k_per = x.shape
    _, n = w_mat.shape
    ch = m // N_DEV
    hn = n // 2

    def body(
        x_ref, w_ref, out_ref,
        mm0, rs_cw, rs_ccw, ss_cw, rs_sem_cw, ss_ccw, rs_sem_ccw,
    ):
        my = lax.axis_index("i")
        left = (my - 1) % N_DEV
        right = (my + 1) % N_DEV

        barrier_sem = pltpu.get_barrier_semaphore()
        for nbr in (left, right):
            pl.semaphore_signal(
                barrier_sem, inc=1,
                device_id=(nbr,), device_id_type=pl.DeviceIdType.MESH,
            )
        pl.semaphore_wait(barrier_sem, 2)

        def mm_chunk(idx):
            out_ref[pl.ds(idx * ch, ch), :] = jnp.dot(
                x_ref[pl.ds(idx * ch, ch), :], w_ref[:, :],
                preferred_element_type=jnp.float32,
            )

        def make_rs(h, cw_send, ccw_send):
            r_cw = pltpu.make_async_remote_copy(
                src_ref=out_ref.at[pl.ds(cw_send * ch, ch), pl.ds(0, hn)],
                dst_ref=rs_cw.at[h],
                send_sem=ss_cw.at[h],
                recv_sem=rs_sem_cw.at[h],
                device_id=(right,),
                device_id_type=pl.DeviceIdType.MESH,
            )
            r_ccw = pltpu.make_async_remote_copy(
                src_ref=out_ref.at[pl.ds(ccw_send * ch, ch), pl.ds(hn, hn)],
                dst_ref=rs_ccw.at[h],
                send_sem=ss_ccw.at[h],
                recv_sem=rs_sem_ccw.at[h],
                device_id=(left,),
                device_id_type=pl.DeviceIdType.MESH,
            )
            return r_cw, r_ccw

        mm0[:, :] = jnp.dot(
            x_ref[pl.ds(my * ch, ch), :], w_ref[:, :],
            preferred_element_type=jnp.float32,
        )
        r_cw = pltpu.make_async_remote_copy(
            src_ref=mm0.at[:, pl.ds(0, hn)],
            dst_ref=rs_cw.at[0],
            send_sem=ss_cw.at[0],
            recv_sem=rs_sem_cw.at[0],
            device_id=(right,),
            device_id_type=pl.DeviceIdType.MESH,
        )
        r_ccw = pltpu.make_async_remote_copy(
            src_ref=mm0.at[:, pl.ds(hn, hn)],
            dst_ref=rs_ccw.at[0],
            send_sem=ss_ccw.at[0],
            recv_sem=rs_sem_ccw.at[0],
            device_id=(left,),
            device_id_type=pl.DeviceIdType.MESH,
        )
        r_cw.start()
        r_ccw.start()
        mm_chunk((my - 1) % N_DEV)
        mm_chunk((my + 1) % N_DEV)
        mm_chunk((my + 2) % N_DEV)

        for h in range(N_DEV - 1):
            cw_recv = (my - h - 1) % N_DEV
            ccw_recv = (my + h + 1) % N_DEV
            r_cw.wait()
            out_ref[pl.ds(cw_recv * ch, ch), pl.ds(0, hn)] = (
                out_ref[pl.ds(cw_recv * ch, ch), pl.ds(0, hn)] + rs_cw[h, :, :]
            )
            r_ccw.wait()
            out_ref[pl.ds(ccw_recv * ch, ch), pl.ds(hn, hn)] = (
                out_ref[pl.ds(ccw_recv * ch, ch), pl.ds(hn, hn)] + rs_ccw[h, :, :]
            )
            if h < N_DEV - 2:
                r_cw, r_ccw = make_rs(h + 1, cw_recv, ccw_recv)
                r_cw.start()
                r_ccw.start()

        own_cw = (my + 1) % N_DEV
        own_ccw = (my - 1) % N_DEV
        out_ref[pl.ds(own_cw * ch, ch), pl.ds(0, hn)] = _gelu(
            out_ref[pl.ds(own_cw * ch, ch), pl.ds(0, hn)]
        )
        out_ref[pl.ds(own_ccw * ch, ch), pl.ds(hn, hn)] = _gelu(
            out_ref[pl.ds(own_ccw * ch, ch), pl.ds(hn, hn)]
        )

        for g in range(N_DEV - 1):
            cw_send = (my + 1 - g) % N_DEV
            ccw_send = (my - 1 + g) % N_DEV
            r_cw = pltpu.make_async_remote_copy(
                src_ref=out_ref.at[pl.ds(cw_send * ch, ch), pl.ds(0, hn)],
                dst_ref=out_ref.at[pl.ds(cw_send * ch, ch), pl.ds(0, hn)],
                send_sem=ss_cw.at[N_DEV - 1 + g],
                recv_sem=rs_sem_cw.at[N_DEV - 1 + g],
                device_id=(right,),
                device_id_type=pl.DeviceIdType.MESH,
            )
            r_ccw = pltpu.make_async_remote_copy(
                src_ref=out_ref.at[pl.ds(ccw_send * ch, ch), pl.ds(hn, hn)],
                dst_ref=out_ref.at[pl.ds(ccw_send * ch, ch), pl.ds(hn, hn)],
                send_sem=ss_ccw.at[N_DEV - 1 + g],
                recv_sem=rs_sem_ccw.at[N_DEV - 1 + g],
                device_id=(left,),
                device_id_type=pl.DeviceIdType.MESH,
            )
            r_cw.start()
            r_ccw.start()
            r_cw.wait()
            r_ccw.wait()

    n_sems = 2 * (N_DEV - 1)
    return pl.pallas_call(
        body,
        out_shape=jax.ShapeDtypeStruct((m, n), jnp.float32),
        in_specs=[
            pl.BlockSpec(memory_space=pltpu.VMEM),
            pl.BlockSpec(memory_space=pltpu.VMEM),
        ],
        out_specs=pl.BlockSpec(memory_space=pltpu.VMEM),
        scratch_shapes=[
            pltpu.VMEM((ch, n), jnp.float32),
            pltpu.VMEM((N_DEV - 1, ch, hn), jnp.float32),
            pltpu.VMEM((N_DEV - 1, ch, hn), jnp.float32),
            pltpu.SemaphoreType.DMA((n_sems,)),
            pltpu.SemaphoreType.DMA((n_sems,)),
            pltpu.SemaphoreType.DMA((n_sems,)),
            pltpu.SemaphoreType.DMA((n_sems,)),
        ],
        compiler_params=pltpu.CompilerParams(
            collective_id=0, vmem_limit_bytes=56 * 1024 * 1024
        ),
    )(x, w_mat)
